# baseline (speedup 1.0000x reference)
import functools

import jax
import jax.numpy as jnp
from jax import lax
from jax.experimental import pallas as pl
from jax.experimental.pallas import tpu as pltpu
from jax.experimental.pallas import tpu_sc as plsc

PAD = 0
SMOOTHING = 0.1
CONFIDENCE = 1.0 - SMOOTHING

N_ROWS = 1024
N_COLS = 100000
LANES = 16
EPS = SMOOTHING / (N_COLS - 2)

NC, NS = 2, 16
NW = NC * NS

TROW, TCOL = 8, 128
R_SC = 512
RPW = R_SC // NW
NBAND = RPW // TROW
CT = 11
CW = CT * TCOL
NFULL = N_COLS // TCOL
NCH = NFULL // CT
TAIL_LO = NFULL * TCOL
NTAIL_SL = (N_COLS - TAIL_LO) // LANES


def _sc_sweep_body(x_hbm, tgt_hbm, out_hbm, tgt_v, xtile_v, x0tile_v,
                   accv, buf0, buf1, semg, sem0, sem1):
    wid = lax.axis_index("s") * NC + lax.axis_index("c")
    base = wid * RPW
    pltpu.sync_copy(tgt_hbm.at[pl.ds(base, RPW)], tgt_v)
    tv = tgt_v[...]

    descs = []
    for k in range(RPW):
        col128 = pl.multiple_of((tv[k] >> 7) << 7, TCOL)
        row8 = pl.multiple_of(base + (k & ~(TROW - 1)), TROW)
        d = pltpu.make_async_copy(
            x_hbm.at[pl.ds(row8, TROW), pl.ds(col128, TCOL)],
            xtile_v.at[k], semg)
        d.start()
        descs.append(d)
    for b in range(NBAND):
        row8 = pl.multiple_of(base + b * TROW, TROW)
        d = pltpu.make_async_copy(
            x_hbm.at[pl.ds(row8, TROW), pl.ds(0, TCOL)],
            x0tile_v.at[b], semg)
        d.start()
        descs.append(d)
    for d in descs:
        d.wait()
    i16 = lax.iota(jnp.int32, 16)
    total = jnp.zeros((16,), jnp.float32)
    for k in range(RPW):
        r = k % TROW
        t_k = tv[k]
        off = pl.multiple_of(((t_k & (TCOL - 1)) >> 4) << 4, LANES)
        xt_slice = xtile_v[k, r, pl.ds(off, LANES)]
        wt = jnp.where(t_k != PAD, jnp.float32(CONFIDENCE - EPS),
                       jnp.float32(0.0))
        total = total + jnp.where(i16 == (t_k & (LANES - 1)),
                                  xt_slice, 0.0) * wt
        x0_slice = x0tile_v[k // TROW, r, pl.ds(0, LANES)]
        w0 = jnp.where(t_k != PAD, jnp.float32(-EPS), jnp.float32(0.0))
        total = total + jnp.where(i16 == 0, x0_slice, 0.0) * w0

    for b in range(NBAND):
        row8 = pl.multiple_of(base + b * TROW, TROW)
        ws = [jnp.where(tv[b * TROW + r] != PAD,
                        jnp.float32(EPS), jnp.float32(0.0))
              for r in range(TROW)]

        def chunk_sum(buf, acc):
            for r in range(TROW):
                def tile_body(ti, a):
                    off = pl.multiple_of(ti * TCOL, TCOL)
                    for sl in range(TCOL // LANES):
                        a = a + buf[r, pl.ds(off + sl * LANES, LANES)]
                    return a
                racc = lax.fori_loop(
                    0, CT, tile_body, jnp.zeros((16,), jnp.float32))
                acc = acc + racc * ws[r]
            return acc

        def start_chunk(ci, buf, sem):
            off = pl.multiple_of(ci * CW, TCOL)
            pltpu.make_async_copy(
                x_hbm.at[pl.ds(row8, TROW), pl.ds(off, CW)], buf, sem
            ).start()

        start_chunk(0, buf0, sem0)
        start_chunk(1, buf1, sem1)

        def pair_body(p, acc):
            i0 = 2 * p
            pltpu.make_async_copy(
                x_hbm.at[pl.ds(row8, TROW), pl.ds(0, CW)], buf0, sem0).wait()
            acc = chunk_sum(buf0, acc)

            @pl.when(i0 + 2 < NCH)
            def _():
                start_chunk(i0 + 2, buf0, sem0)

            pltpu.make_async_copy(
                x_hbm.at[pl.ds(row8, TROW), pl.ds(0, CW)], buf1, sem1).wait()
            acc = chunk_sum(buf1, acc)

            @pl.when(i0 + 3 < NCH)
            def _():
                start_chunk(i0 + 3, buf1, sem1)

            return acc

        total = lax.fori_loop(0, NCH // 2, pair_body, total)
        pltpu.make_async_copy(
            x_hbm.at[pl.ds(row8, TROW), pl.ds(0, CW)], buf0, sem0).wait()
        total = chunk_sum(buf0, total)


    accv[...] = total
    pltpu.sync_copy(accv, out_hbm.at[pl.ds(wid * 16, 16)])


@functools.cache
def _sc_sweep():
    mesh = plsc.VectorSubcoreMesh(
        core_axis_name="c", subcore_axis_name="s", num_cores=NC, num_subcores=NS
    )
    return pl.kernel(
        _sc_sweep_body,
        out_type=jax.ShapeDtypeStruct((NW * 16,), jnp.float32),
        mesh=mesh,
        scratch_types=[
            pltpu.VMEM((RPW,), jnp.int32),
            pltpu.VMEM((RPW, TROW, TCOL), jnp.float32),
            pltpu.VMEM((NBAND, TROW, TCOL), jnp.float32),
            pltpu.VMEM((16,), jnp.float32),
            pltpu.VMEM((TROW, CW), jnp.float32),
            pltpu.VMEM((TROW, CW), jnp.float32),
            pltpu.SemaphoreType.DMA,
            pltpu.SemaphoreType.DMA,
            pltpu.SemaphoreType.DMA,
        ],
    )


BR = 32
TC_OFF = R_SC // BR


def _tc_body(x_ref, t_ref, tail_ref, tsc_ref, out_ref, acc_ref):
    r = pl.program_id(0)
    blk = x_ref[...]
    t = t_ref[...]
    cols = lax.broadcasted_iota(jnp.int32, (BR, N_COLS), 1)
    w = jnp.where(cols == t, jnp.float32(CONFIDENCE), jnp.float32(EPS))
    s = jnp.sum(blk * w, axis=1, keepdims=True)
    row = s - EPS * blk[:, 0:1]
    row = jnp.where(t != PAD, row, 0.0)
    part = jnp.sum(row)

    @pl.when(r == 0)
    def _():
        tblk = tail_ref[...]
        lanes = lax.broadcasted_iota(jnp.int32, (R_SC, TCOL), 1)
        tblk = jnp.where(lanes < N_COLS - TAIL_LO, tblk, 0.0)
        tails = jnp.sum(tblk, axis=1, keepdims=True)
        tsc = tsc_ref[...]
        acc_ref[0] = jnp.sum(
            jnp.where(tsc != PAD, EPS * tails, 0.0))

    acc_ref[0] += part

    @pl.when(r == pl.num_programs(0) - 1)
    def _():
        out_ref[0, 0] = acc_ref[0]


def kernel(x, target):
    target = target.astype(jnp.int32)
    sc_parts = _sc_sweep()(x, target)
    tc_part = pl.pallas_call(
        _tc_body,
        grid=((N_ROWS - R_SC) // BR,),
        in_specs=[
            pl.BlockSpec((BR, N_COLS), lambda r: (r + TC_OFF, 0)),
            pl.BlockSpec((BR, 1), lambda r: (r + TC_OFF, 0)),
            pl.BlockSpec((R_SC, TCOL), lambda r: (0, NFULL)),
            pl.BlockSpec((R_SC, 1), lambda r: (0, 0)),
        ],
        out_specs=pl.BlockSpec(memory_space=pltpu.SMEM),
        out_shape=jax.ShapeDtypeStruct((1, 1), jnp.float32),
        scratch_shapes=[pltpu.SMEM((1,), jnp.float32)],
    )(x, target.reshape(N_ROWS, 1), x, target.reshape(N_ROWS, 1))
    return -(jnp.sum(sc_parts) + tc_part[0, 0]) / N_ROWS
